# baseline (speedup 1.0000x reference)
import functools

import jax
import jax.numpy as jnp
from jax import lax
from jax.experimental import pallas as pl
from jax.experimental.pallas import tpu as pltpu
from jax.experimental.pallas import tpu_sc as plsc

N = 10000
E = 160000
D = 256
DH = 128
NP_ = 10240
EP = 163840
NS = 16
CHUNK = 128
CH_PER_SUB = EP // (NS * CHUNK)
NPH = 5
CPP = CH_PER_SUB // NPH
ROWS_PER_SUB = NP_ // NS
MMB = 1024
FB = 1000

@functools.cache
def _mesh():
    return plsc.VectorSubcoreMesh(core_axis_name="c", subcore_axis_name="s",
                                  num_cores=2, num_subcores=NS)


def _mm_body(x_ref, w_ref, o_ref):
    o_ref[...] = jnp.dot(x_ref[...], w_ref[...],
                         preferred_element_type=jnp.float32)


def _matmul(x_p, W):
    return pl.pallas_call(
        _mm_body,
        grid=(NP_ // MMB,),
        in_specs=[pl.BlockSpec((MMB, D), lambda i: (i, 0)),
                  pl.BlockSpec((D, D), lambda i: (0, 0))],
        out_specs=pl.BlockSpec((MMB, D), lambda i: (i, 0)),
        out_shape=jax.ShapeDtypeStruct((NP_, D), jnp.float32),
    )(x_p, W)


def _deg_sc(dst2):
    @functools.partial(
        pl.kernel,
        out_type=jax.ShapeDtypeStruct((NP_,), jnp.float32),
        mesh=_mesh(),
        scratch_types=[
            pltpu.VMEM((CH_PER_SUB, CHUNK), jnp.int32),
            pltpu.VMEM((CHUNK,), jnp.float32),
            pltpu.VMEM((ROWS_PER_SUB,), jnp.float32),
            pltpu.VMEM_SHARED((NP_,), jnp.float32),
        ],
    )
    def k(dst_hbm, deg_hbm, idx_v, ones_v, zero_v, deg_sp):
        cid = lax.axis_index("c")
        sid = lax.axis_index("s")

        @pl.when(cid == 0)
        def _():
            @pl.loop(0, CHUNK, step=16)
            def _(i):
                ones_v[pl.ds(i, 16)] = jnp.ones((16,), jnp.float32)

            @pl.loop(0, ROWS_PER_SUB, step=16)
            def _(i):
                zero_v[pl.ds(i, 16)] = jnp.zeros((16,), jnp.float32)

            base = sid * ROWS_PER_SUB
            pltpu.sync_copy(zero_v, deg_sp.at[pl.ds(base, ROWS_PER_SUB)])
            plsc.subcore_barrier()
            pltpu.sync_copy(dst_hbm.at[pl.ds(sid * CH_PER_SUB, CH_PER_SUB)],
                            idx_v)

            @pl.loop(0, CH_PER_SUB)
            def _(j):
                pltpu.sync_copy(ones_v, deg_sp.at[idx_v.at[j]], add=True)

            plsc.subcore_barrier()
            pltpu.sync_copy(deg_sp.at[pl.ds(base, ROWS_PER_SUB)],
                            deg_hbm.at[pl.ds(base, ROWS_PER_SUB)])

    return k(dst2)


def _scale_body(h_ref, deg_ref, h2a_ref, h2b_ref, dis_ref):
    dis = lax.rsqrt(deg_ref[...] + 1.0)
    h = h_ref[...]
    h2a_ref[...] = h[:, :DH] * dis
    h2b_ref[...] = h[:, DH:] * dis
    dis_ref[...] = jnp.broadcast_to(dis, (MMB, DH))


def _scale(h_p, deg_col):
    return pl.pallas_call(
        _scale_body,
        grid=(NP_ // MMB,),
        in_specs=[pl.BlockSpec((MMB, D), lambda i: (i, 0)),
                  pl.BlockSpec((MMB, 1), lambda i: (i, 0))],
        out_specs=[pl.BlockSpec((MMB, DH), lambda i: (i, 0)),
                   pl.BlockSpec((MMB, DH), lambda i: (i, 0)),
                   pl.BlockSpec((MMB, DH), lambda i: (i, 0))],
        out_shape=[jax.ShapeDtypeStruct((NP_, DH), jnp.float32),
                   jax.ShapeDtypeStruct((NP_, DH), jnp.float32),
                   jax.ShapeDtypeStruct((NP_, DH), jnp.float32)],
    )(h_p, deg_col)


def _scatter_sc(h2a, h2b, src2, dst2):
    @functools.partial(
        pl.kernel,
        out_type=(jax.ShapeDtypeStruct((NP_, DH), jnp.float32),
                  jax.ShapeDtypeStruct((NP_, DH), jnp.float32)),
        mesh=_mesh(),
        scratch_types=[
            pltpu.VMEM((2, CPP, CHUNK), jnp.int32),
            pltpu.VMEM((2, CPP, CHUNK), jnp.int32),
            pltpu.VMEM((CHUNK, DH), jnp.float32),
            pltpu.VMEM((CHUNK, DH), jnp.float32),
            pltpu.VMEM_SHARED((NP_, DH), jnp.float32),
            pltpu.SemaphoreType.DMA,
            pltpu.SemaphoreType.DMA,
            pltpu.SemaphoreType.DMA,
            pltpu.SemaphoreType.DMA,
            pltpu.SemaphoreType.DMA,
        ],
    )
    def k(h2a_hbm, h2b_hbm, src_hbm, dst_hbm, s0_hbm, s1_hbm,
          isrc, idst, buf0, buf1, acc, gsem0, gsem1, ssem0, ssem1, isem):
        cid = lax.axis_index("c")
        sid = lax.axis_index("s")
        base = sid * ROWS_PER_SUB

        @pl.loop(0, CHUNK)
        def _(r):
            @pl.loop(0, DH, step=16)
            def _(q):
                buf0[r, pl.ds(q, 16)] = jnp.zeros((16,), jnp.float32)

        @pl.loop(0, ROWS_PER_SUB // CHUNK)
        def _(t):
            pltpu.sync_copy(buf0, acc.at[pl.ds(base + t * CHUNK, CHUNK)])

        plsc.subcore_barrier()

        def run(h2_hbm, out_hbm):
            pltpu.sync_copy(src_hbm.at[pl.ds(sid * CH_PER_SUB, CPP)],
                            isrc.at[0])
            pltpu.sync_copy(dst_hbm.at[pl.ds(sid * CH_PER_SUB, CPP)],
                            idst.at[0])
            for p in range(NPH):
                cur = p % 2
                nxt = 1 - cur
                if p + 1 < NPH:
                    pbase = sid * CH_PER_SUB + (p + 1) * CPP
                    pltpu.async_copy(src_hbm.at[pl.ds(pbase, CPP)],
                                     isrc.at[nxt], isem)
                    pltpu.async_copy(dst_hbm.at[pl.ds(pbase, CPP)],
                                     idst.at[nxt], isem)
                s_v = isrc.at[cur]
                d_v = idst.at[cur]
                pltpu.async_copy(h2_hbm.at[s_v.at[0]], buf0, gsem0)
                pltpu.async_copy(h2_hbm.at[s_v.at[1]], buf1, gsem1)

                @pl.loop(0, CPP, step=2)
                def _(j):
                    pltpu.make_async_copy(h2_hbm.at[s_v.at[j]], buf0,
                                          gsem0).wait()
                    pltpu.async_copy(buf0, acc.at[d_v.at[j]], ssem0, add=True)
                    pltpu.make_async_copy(h2_hbm.at[s_v.at[j + 1]], buf1,
                                          gsem1).wait()
                    pltpu.async_copy(buf1, acc.at[d_v.at[j + 1]], ssem1,
                                     add=True)

                    @pl.when(j + 2 < CPP)
                    def _():
                        pltpu.make_async_copy(buf0, acc.at[d_v.at[j]], ssem0).wait()
                        pltpu.async_copy(h2_hbm.at[s_v.at[j + 2]], buf0, gsem0)

                    @pl.when(j + 3 < CPP)
                    def _():
                        pltpu.make_async_copy(buf1, acc.at[d_v.at[j + 1]], ssem1).wait()
                        pltpu.async_copy(h2_hbm.at[s_v.at[j + 3]], buf1, gsem1)

                pltpu.make_async_copy(buf0, acc.at[d_v.at[0]], ssem0).wait()
                pltpu.make_async_copy(buf1, acc.at[d_v.at[1]], ssem1).wait()
                if p + 1 < NPH:
                    pltpu.make_async_copy(src_hbm.at[pl.ds(0, CPP)],
                                          isrc.at[nxt], isem).wait()
                    pltpu.make_async_copy(dst_hbm.at[pl.ds(0, CPP)],
                                          idst.at[nxt], isem).wait()

            plsc.subcore_barrier()
            pltpu.sync_copy(acc.at[pl.ds(base, ROWS_PER_SUB)],
                            out_hbm.at[pl.ds(base, ROWS_PER_SUB)])

        @pl.when(cid == 0)
        def _():
            run(h2a_hbm, s0_hbm)

        @pl.when(cid == 1)
        def _():
            run(h2b_hbm, s1_hbm)

    return k(h2a, h2b, src2, dst2)


def _final_body(s0_ref, s1_ref, h_ref, x_ref, dis_ref, b_ref, g_ref, bt_ref,
                o_ref):
    dis = dis_ref[:, 0:1]
    s = jnp.concatenate([s0_ref[...], s1_ref[...]], axis=1)
    pre = s * dis + h_ref[...] * (dis * dis) + b_ref[...]
    mu = jnp.mean(pre, axis=-1, keepdims=True)
    var = jnp.mean((pre - mu) ** 2, axis=-1, keepdims=True)
    y = (pre - mu) / jnp.sqrt(var + 1e-5) * g_ref[...] + bt_ref[...]
    o_ref[...] = y * 0.5 * (1.0 + lax.erf(y * 0.7071067811865476)) + x_ref[...]


def _final(s0, s1, h_p, x, dis2d, b, gamma, beta):
    vec = pl.BlockSpec((1, D), lambda i: (0, 0))
    return pl.pallas_call(
        _final_body,
        grid=(N // FB,),
        in_specs=[pl.BlockSpec((FB, DH), lambda i: (i, 0)),
                  pl.BlockSpec((FB, DH), lambda i: (i, 0)),
                  pl.BlockSpec((FB, D), lambda i: (i, 0)),
                  pl.BlockSpec((FB, D), lambda i: (i, 0)),
                  pl.BlockSpec((FB, DH), lambda i: (i, 0)),
                  vec, vec, vec],
        out_specs=pl.BlockSpec((FB, D), lambda i: (i, 0)),
        out_shape=jax.ShapeDtypeStruct((N, D), jnp.float32),
    )(s0, s1, h_p, x, dis2d, b, gamma, beta)


def kernel(x, edge_index, W, b, gamma, beta):
    src = edge_index[0].astype(jnp.int32)
    dst = edge_index[1].astype(jnp.int32)
    pad = (jnp.arange(EP - E, dtype=jnp.int32) % (NP_ - N)) + N
    src2 = jnp.concatenate([src, pad]).reshape(EP // CHUNK, CHUNK)
    dst2 = jnp.concatenate([dst, pad]).reshape(EP // CHUNK, CHUNK)
    x_p = jnp.pad(x, ((0, NP_ - N), (0, 0)))

    h_p = _matmul(x_p, W)
    deg = _deg_sc(dst2)
    h2a, h2b, dis2d = _scale(h_p, deg.reshape(NP_, 1))
    s0, s1 = _scatter_sc(h2a, h2b, src2, dst2)
    return _final(s0, s1, h_p, x, dis2d, b.reshape(1, D),
                  gamma.reshape(1, D), beta.reshape(1, D))


# speedup vs baseline: 1.1533x; 1.1533x over previous
import functools

import jax
import jax.numpy as jnp
from jax import lax
from jax.experimental import pallas as pl
from jax.experimental.pallas import tpu as pltpu
from jax.experimental.pallas import tpu_sc as plsc

N = 10000
E = 160000
D = 256
DH = 128
NP_ = 10240
EP = 163840
NS = 16
CHUNK = 128
CH_PER_SUB = EP // (NS * CHUNK)
NPH = 5
CPP = CH_PER_SUB // NPH
ROWS_PER_SUB = NP_ // NS
MMB = 1024
FB = 1000

@functools.cache
def _mesh():
    return plsc.VectorSubcoreMesh(core_axis_name="c", subcore_axis_name="s",
                                  num_cores=2, num_subcores=NS)


def _mm_body(x_ref, w_ref, o_ref):
    o_ref[...] = jnp.dot(x_ref[...], w_ref[...],
                         preferred_element_type=jnp.float32)


def _matmul(x_p, W):
    return pl.pallas_call(
        _mm_body,
        grid=(NP_ // MMB,),
        in_specs=[pl.BlockSpec((MMB, D), lambda i: (i, 0)),
                  pl.BlockSpec((D, D), lambda i: (0, 0))],
        out_specs=pl.BlockSpec((MMB, D), lambda i: (i, 0)),
        out_shape=jax.ShapeDtypeStruct((NP_, D), jnp.float32),
    )(x_p, W)


def _deg_sc(dst2):
    @functools.partial(
        pl.kernel,
        out_type=jax.ShapeDtypeStruct((NP_,), jnp.float32),
        mesh=_mesh(),
        scratch_types=[
            pltpu.VMEM((CH_PER_SUB, CHUNK), jnp.int32),
            pltpu.VMEM((CHUNK,), jnp.float32),
            pltpu.VMEM((ROWS_PER_SUB,), jnp.float32),
            pltpu.VMEM_SHARED((NP_,), jnp.float32),
        ],
    )
    def k(dst_hbm, deg_hbm, idx_v, ones_v, zero_v, deg_sp):
        cid = lax.axis_index("c")
        sid = lax.axis_index("s")

        @pl.when(cid == 0)
        def _():
            @pl.loop(0, CHUNK, step=16)
            def _(i):
                ones_v[pl.ds(i, 16)] = jnp.ones((16,), jnp.float32)

            @pl.loop(0, ROWS_PER_SUB, step=16)
            def _(i):
                zero_v[pl.ds(i, 16)] = jnp.zeros((16,), jnp.float32)

            base = sid * ROWS_PER_SUB
            pltpu.sync_copy(zero_v, deg_sp.at[pl.ds(base, ROWS_PER_SUB)])
            plsc.subcore_barrier()
            pltpu.sync_copy(dst_hbm.at[pl.ds(sid * CH_PER_SUB, CH_PER_SUB)],
                            idx_v)

            @pl.loop(0, CH_PER_SUB)
            def _(j):
                pltpu.sync_copy(ones_v, deg_sp.at[idx_v.at[j]], add=True)

            plsc.subcore_barrier()
            pltpu.sync_copy(deg_sp.at[pl.ds(base, ROWS_PER_SUB)],
                            deg_hbm.at[pl.ds(base, ROWS_PER_SUB)])

    return k(dst2)


def _scale_body(h_ref, deg_ref, h2a_ref, h2b_ref):
    dis = lax.rsqrt(deg_ref[...] + 1.0)
    h = h_ref[...]
    h2a_ref[...] = h[:, :DH] * dis
    h2b_ref[...] = h[:, DH:] * dis


def _scale(h_p, deg_col):
    return pl.pallas_call(
        _scale_body,
        grid=(NP_ // MMB,),
        in_specs=[pl.BlockSpec((MMB, D), lambda i: (i, 0)),
                  pl.BlockSpec((MMB, 1), lambda i: (i, 0))],
        out_specs=[pl.BlockSpec((MMB, DH), lambda i: (i, 0)),
                   pl.BlockSpec((MMB, DH), lambda i: (i, 0))],
        out_shape=[jax.ShapeDtypeStruct((NP_, DH), jnp.float32),
                   jax.ShapeDtypeStruct((NP_, DH), jnp.float32)],
    )(h_p, deg_col)


def _scatter_sc(h2a, h2b, src2, dst2):
    @functools.partial(
        pl.kernel,
        out_type=(jax.ShapeDtypeStruct((NP_, DH), jnp.float32),
                  jax.ShapeDtypeStruct((NP_, DH), jnp.float32)),
        mesh=_mesh(),
        scratch_types=[
            pltpu.VMEM((CPP, CHUNK), jnp.int32),
            pltpu.VMEM((CPP, CHUNK), jnp.int32),
            pltpu.VMEM((CHUNK, DH), jnp.float32),
            pltpu.VMEM((CHUNK, DH), jnp.float32),
            pltpu.VMEM_SHARED((NP_, DH), jnp.float32),
            pltpu.SemaphoreType.DMA,
            pltpu.SemaphoreType.DMA,
        ],
    )
    def k(h2a_hbm, h2b_hbm, src_hbm, dst_hbm, s0_hbm, s1_hbm,
          isrc, idst, buf0, buf1, acc, sem0, sem1):
        cid = lax.axis_index("c")
        sid = lax.axis_index("s")
        base = sid * ROWS_PER_SUB

        @pl.loop(0, CHUNK)
        def _(r):
            @pl.loop(0, DH, step=16)
            def _(q):
                buf0[r, pl.ds(q, 16)] = jnp.zeros((16,), jnp.float32)

        @pl.loop(0, ROWS_PER_SUB // CHUNK)
        def _(t):
            pltpu.sync_copy(buf0, acc.at[pl.ds(base + t * CHUNK, CHUNK)])

        plsc.subcore_barrier()

        def run(h2_hbm, out_hbm):
            @pl.loop(0, NPH)
            def _(p):
                pbase = sid * CH_PER_SUB + p * CPP
                pltpu.sync_copy(src_hbm.at[pl.ds(pbase, CPP)], isrc)
                pltpu.sync_copy(dst_hbm.at[pl.ds(pbase, CPP)], idst)
                pltpu.async_copy(h2_hbm.at[isrc.at[0]], buf0, sem0)
                pltpu.async_copy(h2_hbm.at[isrc.at[1]], buf1, sem1)

                @pl.loop(0, CPP, step=2)
                def _(j):
                    pltpu.make_async_copy(h2_hbm.at[isrc.at[j]], buf0,
                                          sem0).wait()
                    pltpu.sync_copy(buf0, acc.at[idst.at[j]], add=True)

                    @pl.when(j + 2 < CPP)
                    def _():
                        pltpu.async_copy(h2_hbm.at[isrc.at[j + 2]], buf0, sem0)

                    pltpu.make_async_copy(h2_hbm.at[isrc.at[j + 1]], buf1,
                                          sem1).wait()
                    pltpu.sync_copy(buf1, acc.at[idst.at[j + 1]], add=True)

                    @pl.when(j + 3 < CPP)
                    def _():
                        pltpu.async_copy(h2_hbm.at[isrc.at[j + 3]], buf1, sem1)

            plsc.subcore_barrier()
            pltpu.sync_copy(acc.at[pl.ds(base, ROWS_PER_SUB)],
                            out_hbm.at[pl.ds(base, ROWS_PER_SUB)])

        @pl.when(cid == 0)
        def _():
            run(h2a_hbm, s0_hbm)

        @pl.when(cid == 1)
        def _():
            run(h2b_hbm, s1_hbm)

    return k(h2a, h2b, src2, dst2)


def _final_body(s0_ref, s1_ref, h_ref, x_ref, deg_ref, b_ref, g_ref, bt_ref,
                o_ref):
    dis = lax.rsqrt(deg_ref[...] + 1.0)
    s = jnp.concatenate([s0_ref[...], s1_ref[...]], axis=1)
    pre = s * dis + h_ref[...] * (dis * dis) + b_ref[...]
    mu = jnp.mean(pre, axis=-1, keepdims=True)
    var = jnp.mean((pre - mu) ** 2, axis=-1, keepdims=True)
    y = (pre - mu) / jnp.sqrt(var + 1e-5) * g_ref[...] + bt_ref[...]
    o_ref[...] = y * 0.5 * (1.0 + lax.erf(y * 0.7071067811865476)) + x_ref[...]


def _final(s0, s1, h_p, x, deg_col, b, gamma, beta):
    vec = pl.BlockSpec((1, D), lambda i: (0, 0))
    return pl.pallas_call(
        _final_body,
        grid=(N // FB,),
        in_specs=[pl.BlockSpec((FB, DH), lambda i: (i, 0)),
                  pl.BlockSpec((FB, DH), lambda i: (i, 0)),
                  pl.BlockSpec((FB, D), lambda i: (i, 0)),
                  pl.BlockSpec((FB, D), lambda i: (i, 0)),
                  pl.BlockSpec((FB, 1), lambda i: (i, 0)),
                  vec, vec, vec],
        out_specs=pl.BlockSpec((FB, D), lambda i: (i, 0)),
        out_shape=jax.ShapeDtypeStruct((N, D), jnp.float32),
    )(s0, s1, h_p, x, deg_col, b, gamma, beta)


def kernel(x, edge_index, W, b, gamma, beta):
    src = edge_index[0].astype(jnp.int32)
    dst = edge_index[1].astype(jnp.int32)
    pad = (jnp.arange(EP - E, dtype=jnp.int32) % (NP_ - N)) + N
    src2 = jnp.concatenate([src, pad]).reshape(EP // CHUNK, CHUNK)
    dst2 = jnp.concatenate([dst, pad]).reshape(EP // CHUNK, CHUNK)
    x_p = jnp.pad(x, ((0, NP_ - N), (0, 0)))

    h_p = _matmul(x_p, W)
    deg = _deg_sc(dst2)
    deg_col = deg.reshape(NP_, 1)
    h2a, h2b = _scale(h_p, deg_col)
    s0, s1 = _scatter_sc(h2a, h2b, src2, dst2)
    return _final(s0, s1, h_p, x, deg_col, b.reshape(1, D),
                  gamma.reshape(1, D), beta.reshape(1, D))


# speedup vs baseline: 1.2040x; 1.0440x over previous
import functools

import jax
import jax.numpy as jnp
from jax import lax
from jax.experimental import pallas as pl
from jax.experimental.pallas import tpu as pltpu
from jax.experimental.pallas import tpu_sc as plsc

N = 10000
E = 160000
D = 256
DH = 128
NP_ = 10240
EP = 163840
NS = 16
CHUNK = 128
CH_PER_SUB = EP // (NS * CHUNK)
NPH = 5
CPP = CH_PER_SUB // NPH
ROWS_PER_SUB = NP_ // NS
MMB = 1000
FB = 1000

@functools.cache
def _mesh():
    return plsc.VectorSubcoreMesh(core_axis_name="c", subcore_axis_name="s",
                                  num_cores=2, num_subcores=NS)


def _mm_body(x_ref, deg_ref, w_ref, h2a_ref, h2b_ref):
    dis = lax.rsqrt(deg_ref[...] + 1.0)
    h2 = jnp.dot(x_ref[...], w_ref[...],
                 preferred_element_type=jnp.float32) * dis
    h2a_ref[...] = h2[:, :DH]
    h2b_ref[...] = h2[:, DH:]


def _matmul_scale(x, deg_col, W):
    return pl.pallas_call(
        _mm_body,
        grid=(N // MMB,),
        in_specs=[pl.BlockSpec((MMB, D), lambda i: (i, 0)),
                  pl.BlockSpec((MMB, 1), lambda i: (i, 0)),
                  pl.BlockSpec((D, D), lambda i: (0, 0))],
        out_specs=[pl.BlockSpec((MMB, DH), lambda i: (i, 0)),
                   pl.BlockSpec((MMB, DH), lambda i: (i, 0))],
        out_shape=[jax.ShapeDtypeStruct((NP_, DH), jnp.float32),
                   jax.ShapeDtypeStruct((NP_, DH), jnp.float32)],
    )(x, deg_col, W)


def _deg_sc(dst2):
    @functools.partial(
        pl.kernel,
        out_type=jax.ShapeDtypeStruct((NP_,), jnp.float32),
        mesh=_mesh(),
        scratch_types=[
            pltpu.VMEM((CH_PER_SUB, CHUNK), jnp.int32),
            pltpu.VMEM((CHUNK,), jnp.float32),
            pltpu.VMEM((ROWS_PER_SUB,), jnp.float32),
            pltpu.VMEM_SHARED((NP_,), jnp.float32),
        ],
    )
    def k(dst_hbm, deg_hbm, idx_v, ones_v, zero_v, deg_sp):
        cid = lax.axis_index("c")
        sid = lax.axis_index("s")

        @pl.when(cid == 0)
        def _():
            @pl.loop(0, CHUNK, step=16)
            def _(i):
                ones_v[pl.ds(i, 16)] = jnp.ones((16,), jnp.float32)

            @pl.loop(0, ROWS_PER_SUB, step=16)
            def _(i):
                zero_v[pl.ds(i, 16)] = jnp.zeros((16,), jnp.float32)

            base = sid * ROWS_PER_SUB
            pltpu.sync_copy(zero_v, deg_sp.at[pl.ds(base, ROWS_PER_SUB)])
            plsc.subcore_barrier()
            pltpu.sync_copy(dst_hbm.at[pl.ds(sid * CH_PER_SUB, CH_PER_SUB)],
                            idx_v)

            @pl.loop(0, CH_PER_SUB)
            def _(j):
                pltpu.sync_copy(ones_v, deg_sp.at[idx_v.at[j]], add=True)

            plsc.subcore_barrier()
            pltpu.sync_copy(deg_sp.at[pl.ds(base, ROWS_PER_SUB)],
                            deg_hbm.at[pl.ds(base, ROWS_PER_SUB)])

    return k(dst2)


def _scatter_sc(h2a, h2b, src2, dst2):
    @functools.partial(
        pl.kernel,
        out_type=(jax.ShapeDtypeStruct((NP_, DH), jnp.float32),
                  jax.ShapeDtypeStruct((NP_, DH), jnp.float32)),
        mesh=_mesh(),
        scratch_types=[
            pltpu.VMEM((CPP, CHUNK), jnp.int32),
            pltpu.VMEM((CPP, CHUNK), jnp.int32),
            pltpu.VMEM((CHUNK, DH), jnp.float32),
            pltpu.VMEM((CHUNK, DH), jnp.float32),
            pltpu.VMEM_SHARED((NP_, DH), jnp.float32),
            pltpu.SemaphoreType.DMA,
            pltpu.SemaphoreType.DMA,
        ],
    )
    def k(h2a_hbm, h2b_hbm, src_hbm, dst_hbm, s0_hbm, s1_hbm,
          isrc, idst, buf0, buf1, acc, sem0, sem1):
        cid = lax.axis_index("c")
        sid = lax.axis_index("s")
        base = sid * ROWS_PER_SUB

        @pl.loop(0, CHUNK)
        def _(r):
            @pl.loop(0, DH, step=16)
            def _(q):
                buf0[r, pl.ds(q, 16)] = jnp.zeros((16,), jnp.float32)

        @pl.loop(0, ROWS_PER_SUB // CHUNK)
        def _(t):
            pltpu.sync_copy(buf0, acc.at[pl.ds(base + t * CHUNK, CHUNK)])

        plsc.subcore_barrier()

        def run(h2_hbm, out_hbm):
            @pl.loop(0, NPH)
            def _(p):
                pbase = sid * CH_PER_SUB + p * CPP
                pltpu.sync_copy(src_hbm.at[pl.ds(pbase, CPP)], isrc)
                pltpu.sync_copy(dst_hbm.at[pl.ds(pbase, CPP)], idst)
                pltpu.async_copy(h2_hbm.at[isrc.at[0]], buf0, sem0)
                pltpu.async_copy(h2_hbm.at[isrc.at[1]], buf1, sem1)

                @pl.loop(0, CPP, step=2)
                def _(j):
                    pltpu.make_async_copy(h2_hbm.at[isrc.at[j]], buf0,
                                          sem0).wait()
                    pltpu.sync_copy(buf0, acc.at[idst.at[j]], add=True)

                    @pl.when(j + 2 < CPP)
                    def _():
                        pltpu.async_copy(h2_hbm.at[isrc.at[j + 2]], buf0, sem0)

                    pltpu.make_async_copy(h2_hbm.at[isrc.at[j + 1]], buf1,
                                          sem1).wait()
                    pltpu.sync_copy(buf1, acc.at[idst.at[j + 1]], add=True)

                    @pl.when(j + 3 < CPP)
                    def _():
                        pltpu.async_copy(h2_hbm.at[isrc.at[j + 3]], buf1, sem1)

            plsc.subcore_barrier()
            pltpu.sync_copy(acc.at[pl.ds(base, ROWS_PER_SUB)],
                            out_hbm.at[pl.ds(base, ROWS_PER_SUB)])

        @pl.when(cid == 0)
        def _():
            run(h2a_hbm, s0_hbm)

        @pl.when(cid == 1)
        def _():
            run(h2b_hbm, s1_hbm)

    return k(h2a, h2b, src2, dst2)


def _final_body(s0_ref, s1_ref, h2a_ref, h2b_ref, x_ref, deg_ref, b_ref,
                g_ref, bt_ref, o_ref):
    dis = lax.rsqrt(deg_ref[...] + 1.0)
    pre = jnp.concatenate([s0_ref[...] + h2a_ref[...],
                           s1_ref[...] + h2b_ref[...]], axis=1)
    pre = pre * dis + b_ref[...]
    mu = jnp.mean(pre, axis=-1, keepdims=True)
    var = jnp.mean((pre - mu) ** 2, axis=-1, keepdims=True)
    y = (pre - mu) / jnp.sqrt(var + 1e-5) * g_ref[...] + bt_ref[...]
    o_ref[...] = y * 0.5 * (1.0 + lax.erf(y * 0.7071067811865476)) + x_ref[...]


def _final(s0, s1, h2a, h2b, x, deg_col, b, gamma, beta):
    vec = pl.BlockSpec((1, D), lambda i: (0, 0))
    half = pl.BlockSpec((FB, DH), lambda i: (i, 0))
    return pl.pallas_call(
        _final_body,
        grid=(N // FB,),
        in_specs=[half, half, half, half,
                  pl.BlockSpec((FB, D), lambda i: (i, 0)),
                  pl.BlockSpec((FB, 1), lambda i: (i, 0)),
                  vec, vec, vec],
        out_specs=pl.BlockSpec((FB, D), lambda i: (i, 0)),
        out_shape=jax.ShapeDtypeStruct((N, D), jnp.float32),
    )(s0, s1, h2a, h2b, x, deg_col, b, gamma, beta)


def kernel(x, edge_index, W, b, gamma, beta):
    src = edge_index[0].astype(jnp.int32)
    dst = edge_index[1].astype(jnp.int32)
    pad = (jnp.arange(EP - E, dtype=jnp.int32) % (NP_ - N)) + N
    src2 = jnp.concatenate([src, pad]).reshape(EP // CHUNK, CHUNK)
    dst2 = jnp.concatenate([dst, pad]).reshape(EP // CHUNK, CHUNK)

    deg = _deg_sc(dst2)
    deg_col = deg.reshape(NP_, 1)
    h2a, h2b = _matmul_scale(x, deg_col, W)
    s0, s1 = _scatter_sc(h2a, h2b, src2, dst2)
    return _final(s0, s1, h2a, h2b, x, deg_col, b.reshape(1, D),
                  gamma.reshape(1, D), beta.reshape(1, D))
